# Initial kernel scaffold; baseline (speedup 1.0000x reference)
import functools

import jax
import jax.numpy as jnp
from jax import lax
from jax.experimental import pallas as pl
from jax.experimental.pallas import tpu as pltpu
from jax.experimental.pallas import tpu_sc as plsc

VOCAB = 100000
EMBED_DIM = 50
BATCH = 4096
SEQ = 50

_B = BATCH * SEQ
_NC, _NS = 2, 16
_NW = _NC * _NS
_CHUNK = 128
_PER_W = _B // _NW
_NCHUNK = _PER_W // _CHUNK


def _gather_body(table_hbm, idx_hbm, out_hbm, idx_v, rows_a, rows_b, sem_a, sem_b):
    wid = lax.axis_index("c") * _NS + lax.axis_index("s")
    row_base = wid * _PER_W
    pltpu.sync_copy(idx_hbm.at[pl.ds(wid * _NCHUNK, _NCHUNK)], idx_v)

    bufs = (rows_a, rows_b)
    sems = (sem_a, sem_b)
    pltpu.async_copy(table_hbm.at[idx_v.at[0]], bufs[0], sems[0])
    for j in range(_NCHUNK):
        nxt = j + 1
        if nxt < _NCHUNK:
            pltpu.async_copy(
                table_hbm.at[idx_v.at[nxt]], bufs[nxt % 2], sems[nxt % 2]
            )
        pltpu.make_async_copy(
            table_hbm.at[idx_v.at[j]], bufs[j % 2], sems[j % 2]
        ).wait()
        pltpu.sync_copy(
            bufs[j % 2], out_hbm.at[pl.ds(row_base + j * _CHUNK, _CHUNK)]
        )


@jax.jit
def _embed_gather(table, idx2d):
    k = functools.partial(
        pl.kernel,
        out_type=jax.ShapeDtypeStruct((_B, EMBED_DIM), jnp.float32),
        mesh=plsc.VectorSubcoreMesh(core_axis_name="c", subcore_axis_name="s"),
        scratch_types=[
            pltpu.VMEM((_NCHUNK, _CHUNK), jnp.int32),
            pltpu.VMEM((_CHUNK, EMBED_DIM), jnp.float32),
            pltpu.VMEM((_CHUNK, EMBED_DIM), jnp.float32),
            pltpu.SemaphoreType.DMA,
            pltpu.SemaphoreType.DMA,
        ],
    )(_gather_body)
    return k(table, idx2d)


def kernel(indices, table):
    idx2d = indices.reshape(_NW * _NCHUNK, _CHUNK)
    out = _embed_gather(table, idx2d)
    return out.reshape(BATCH, SEQ, EMBED_DIM)


# speedup vs baseline: 2.7204x; 2.7204x over previous
import functools

import jax
import jax.numpy as jnp
from jax import lax
from jax.experimental import pallas as pl
from jax.experimental.pallas import tpu as pltpu
from jax.experimental.pallas import tpu_sc as plsc

VOCAB = 100000
EMBED_DIM = 50
PAD_DIM = 64
BATCH = 4096
SEQ = 50

_B = BATCH * SEQ
_NC, _NS = 2, 16
_NW = _NC * _NS
_CHUNK = 128
_PER_W = _B // _NW
_NCHUNK = _PER_W // _CHUNK


def _gather_body(table_hbm, idx_hbm, out_hbm, idx_v, rows_a, rows_b, sem_a, sem_b):
    wid = lax.axis_index("c") * _NS + lax.axis_index("s")
    row_base = wid * _PER_W
    pltpu.sync_copy(idx_hbm.at[wid], idx_v)

    bufs = (rows_a, rows_b)
    sems = (sem_a, sem_b)
    pltpu.async_copy(table_hbm.at[idx_v.at[0]], bufs[0], sems[0])
    for j in range(_NCHUNK):
        nxt = j + 1
        if nxt < _NCHUNK:
            pltpu.async_copy(
                table_hbm.at[idx_v.at[nxt]], bufs[nxt % 2], sems[nxt % 2]
            )
        pltpu.make_async_copy(
            table_hbm.at[idx_v.at[j]], bufs[j % 2], sems[j % 2]
        ).wait()
        pltpu.sync_copy(
            bufs[j % 2],
            out_hbm.at[pl.ds(row_base + j * _CHUNK, _CHUNK)],
        )


@jax.jit
def _embed_gather(table_padded, idx3d):
    k = functools.partial(
        pl.kernel,
        out_type=jax.ShapeDtypeStruct((_B, PAD_DIM), jnp.float32),
        mesh=plsc.VectorSubcoreMesh(core_axis_name="c", subcore_axis_name="s"),
        scratch_types=[
            pltpu.VMEM((_NCHUNK, _CHUNK), jnp.int32),
            pltpu.VMEM((_CHUNK, PAD_DIM), jnp.float32),
            pltpu.VMEM((_CHUNK, PAD_DIM), jnp.float32),
            pltpu.SemaphoreType.DMA,
            pltpu.SemaphoreType.DMA,
        ],
        compiler_params=pltpu.CompilerParams(use_tc_tiling_on_sc=False),
    )(_gather_body)
    return k(table_padded, idx3d)


def kernel(indices, table):
    table_padded = jnp.pad(table, ((0, 0), (0, PAD_DIM - EMBED_DIM)))
    idx3d = indices.reshape(_NW, _NCHUNK, _CHUNK)
    out = _embed_gather(table_padded, idx3d)
    return out[:, :EMBED_DIM].reshape(BATCH, SEQ, EMBED_DIM)


# speedup vs baseline: 2.7221x; 1.0007x over previous
import functools

import jax
import jax.numpy as jnp
from jax import lax
from jax.experimental import pallas as pl
from jax.experimental.pallas import tpu as pltpu
from jax.experimental.pallas import tpu_sc as plsc

VOCAB = 100000
EMBED_DIM = 50
PAD_DIM = 64
BATCH = 4096
SEQ = 50

_B = BATCH * SEQ
_NC, _NS = 2, 16
_NW = _NC * _NS
_CHUNK = 128
_PER_W = _B // _NW
_NCHUNK = _PER_W // _CHUNK


def _gather_body(table_hbm, idx_hbm, out_hbm, idx_v, rows_a, rows_b, sem_a, sem_b):
    wid = lax.axis_index("c") * _NS + lax.axis_index("s")
    row_base = wid * _PER_W
    pltpu.sync_copy(idx_hbm.at[pl.ds(row_base, _PER_W)], idx_v)

    bufs = (rows_a, rows_b)
    sems = (sem_a, sem_b)
    pltpu.async_copy(table_hbm.at[idx_v.at[pl.ds(0, _CHUNK)]], bufs[0], sems[0])
    for j in range(_NCHUNK):
        nxt = j + 1
        if nxt < _NCHUNK:
            pltpu.async_copy(
                table_hbm.at[idx_v.at[pl.ds(nxt * _CHUNK, _CHUNK)]],
                bufs[nxt % 2],
                sems[nxt % 2],
            )
        pltpu.make_async_copy(
            table_hbm.at[idx_v.at[pl.ds(j * _CHUNK, _CHUNK)]],
            bufs[j % 2],
            sems[j % 2],
        ).wait()
        pltpu.sync_copy(
            bufs[j % 2],
            out_hbm.at[pl.ds(row_base + j * _CHUNK, _CHUNK)],
        )


@jax.jit
def _embed_gather(table_padded, idx_flat):
    k = functools.partial(
        pl.kernel,
        out_type=jax.ShapeDtypeStruct((_B, PAD_DIM), jnp.float32),
        mesh=plsc.VectorSubcoreMesh(core_axis_name="c", subcore_axis_name="s"),
        scratch_types=[
            pltpu.VMEM((_PER_W,), jnp.int32),
            pltpu.VMEM((_CHUNK, PAD_DIM), jnp.float32),
            pltpu.VMEM((_CHUNK, PAD_DIM), jnp.float32),
            pltpu.SemaphoreType.DMA,
            pltpu.SemaphoreType.DMA,
        ],
        compiler_params=pltpu.CompilerParams(use_tc_tiling_on_sc=False),
    )(_gather_body)
    return k(table_padded, idx_flat)


def kernel(indices, table):
    table_padded = jnp.pad(table, ((0, 0), (0, PAD_DIM - EMBED_DIM)))
    idx_flat = indices.reshape(_B)
    out = _embed_gather(table_padded, idx_flat)
    return out[:, :EMBED_DIM].reshape(BATCH, SEQ, EMBED_DIM)
